# baseline (speedup 1.0000x reference)
import functools

import jax
import jax.numpy as jnp
from jax import lax
from jax.experimental import pallas as pl
from jax.experimental.pallas import tpu as pltpu
from jax.experimental.pallas import tpu_sc as plsc

_L = 16


@functools.lru_cache(maxsize=None)
def _build_gather(n_rows: int, width_j: int, s_pad: int, b_per_w: int,
                  n_chunk: int, chunk: int, num_cores: int):
    mesh = plsc.VectorSubcoreMesh(core_axis_name="c", subcore_axis_name="s")

    @functools.partial(
        pl.kernel,
        mesh=mesh,
        compiler_params=pltpu.CompilerParams(use_tc_tiling_on_sc=False),
        out_type=(
            jax.ShapeDtypeStruct((s_pad,), jnp.float32),
            jax.ShapeDtypeStruct((4, s_pad), jnp.float32),
            jax.ShapeDtypeStruct((s_pad,), jnp.float32),
            jax.ShapeDtypeStruct((s_pad, width_j), jnp.float32),
        ),
        scratch_types=[
            pltpu.VMEM((b_per_w,), jnp.int32),
            pltpu.VMEM((b_per_w,), jnp.int32),
            pltpu.VMEM((n_chunk, chunk), jnp.int32),
            pltpu.VMEM((4 * n_chunk, chunk), jnp.int32),
            pltpu.VMEM((b_per_w,), jnp.float32),
            pltpu.VMEM((4, b_per_w), jnp.float32),
            pltpu.VMEM((b_per_w,), jnp.float32),
            pltpu.VMEM((b_per_w, width_j), jnp.float32),
            pltpu.SemaphoreType.DMA,
        ],
    )
    def gather_kernel(bcol_hbm, xcol_hbm, boxes_hbm, scores_hbm, joints_hbm,
                      bidx_out, boxes_out, scores_out, joints_out,
                      bcol_v, xcol_v, idxf_v, idxb_v, bidx_v, boxes_v,
                      scores_v, joints_v, sem):
        wid = lax.axis_index("s") * num_cores + lax.axis_index("c")
        base = wid * b_per_w
        pltpu.sync_copy(bcol_hbm.at[pl.ds(base, b_per_w)], bcol_v)
        pltpu.sync_copy(xcol_hbm.at[pl.ds(base, b_per_w)], xcol_v)
        for i in range(b_per_w // _L):
            bvec = bcol_v[pl.ds(i * _L, _L)]
            xvec = xcol_v[pl.ds(i * _L, _L)]
            flat = bvec * n_rows + xvec
            j, c = (i * _L) // chunk, (i * _L) % chunk
            idxf_v[j, pl.ds(c, _L)] = flat
            flat4 = flat * 4
            for col in range(4):
                idxb_v[col * n_chunk + j, pl.ds(c, _L)] = flat4 + col
            bidx_v[pl.ds(i * _L, _L)] = bvec.astype(jnp.float32)
        copies = []
        for j in range(n_chunk):
            copies.append(pltpu.async_copy(
                scores_hbm.at[idxf_v.at[j]],
                scores_v.at[pl.ds(j * chunk, chunk)], sem))
            for col in range(4):
                copies.append(pltpu.async_copy(
                    boxes_hbm.at[idxb_v.at[col * n_chunk + j]],
                    boxes_v.at[col, pl.ds(j * chunk, chunk)], sem))
        for k in range(b_per_w):
            copies.append(pltpu.async_copy(
                joints_hbm.at[idxf_v.at[k // chunk, pl.ds(k % chunk, 1)]],
                joints_v.at[pl.ds(k, 1)], sem))
        for cp in copies:
            cp.wait()
        pltpu.sync_copy(bidx_v, bidx_out.at[pl.ds(base, b_per_w)])
        pltpu.sync_copy(boxes_v, boxes_out.at[:, pl.ds(base, b_per_w)])
        pltpu.sync_copy(scores_v, scores_out.at[pl.ds(base, b_per_w)])
        pltpu.sync_copy(joints_v, joints_out.at[pl.ds(base, b_per_w)])

    return gather_kernel


def kernel(pred_boxes, pred_scores, pred_joints, selected_indexes):
    b, n = pred_boxes.shape[0], pred_boxes.shape[1]
    s = selected_indexes.shape[0]
    width_j = pred_joints.shape[2] * pred_joints.shape[3]

    info = plsc.get_sparse_core_info()
    nw = info.num_cores * info.num_subcores
    chunk = 80
    n_chunk = 2
    b_per_w = n_chunk * chunk
    s_pad = nw * b_per_w

    boxes1d = pred_boxes.reshape(b * n * 4)
    scores1d = pred_scores.reshape(b * n)
    joints2d = pred_joints.reshape(b * n, width_j)
    bcol = jnp.zeros((s_pad,), jnp.int32).at[:s].set(selected_indexes[:, 0])
    xcol = jnp.zeros((s_pad,), jnp.int32).at[:s].set(selected_indexes[:, 2])

    fn = _build_gather(n, width_j, s_pad, b_per_w, n_chunk, chunk,
                       info.num_cores)
    bidx, boxes_t, scores, joints = fn(bcol, xcol, boxes1d, scores1d, joints2d)
    return jnp.concatenate(
        [bidx[:s, None], boxes_t[:, :s].T, scores[:s, None], joints[:s]],
        axis=1)


# speedup vs baseline: 5.0240x; 5.0240x over previous
import functools

import jax
import jax.numpy as jnp
from jax import lax
from jax.experimental import pallas as pl
from jax.experimental.pallas import tpu as pltpu
from jax.experimental.pallas import tpu_sc as plsc

_L = 16


@functools.lru_cache(maxsize=None)
def _build_gather(n_rows: int, width_j: int, s_pad: int, b_per_w: int,
                  n_chunk: int, chunk: int, num_cores: int):
    n_cols = 1 + 4 + 1 + width_j

    mesh = plsc.VectorSubcoreMesh(core_axis_name="c", subcore_axis_name="s")

    @functools.partial(
        pl.kernel,
        mesh=mesh,
        compiler_params=pltpu.CompilerParams(use_tc_tiling_on_sc=False),
        out_type=jax.ShapeDtypeStruct((n_cols, s_pad), jnp.float32),
        scratch_types=[
            pltpu.VMEM((b_per_w,), jnp.int32),
            pltpu.VMEM((b_per_w,), jnp.int32),
            pltpu.VMEM((n_chunk, chunk), jnp.int32),
            pltpu.VMEM((n_cols, b_per_w), jnp.float32),
            pltpu.SemaphoreType.DMA,
        ],
    )
    def gather_kernel(bcol_hbm, xcol_hbm, boxes_hbm, scores_hbm, joints_hbm,
                      out_hbm, bcol_v, xcol_v, idxf_v, out_v, sem):
        wid = lax.axis_index("s") * num_cores + lax.axis_index("c")
        base = wid * b_per_w
        pltpu.sync_copy(bcol_hbm.at[pl.ds(base, b_per_w)], bcol_v)
        pltpu.sync_copy(xcol_hbm.at[pl.ds(base, b_per_w)], xcol_v)
        for i in range(b_per_w // _L):
            bvec = bcol_v[pl.ds(i * _L, _L)]
            xvec = xcol_v[pl.ds(i * _L, _L)]
            flat = bvec * n_rows + xvec
            j, c = (i * _L) // chunk, (i * _L) % chunk
            idxf_v[j, pl.ds(c, _L)] = flat
            out_v[0, pl.ds(i * _L, _L)] = bvec.astype(jnp.float32)
        copies = []
        for j in range(n_chunk):
            idxs = idxf_v.at[j]
            dst = pl.ds(j * chunk, chunk)
            copies.append(pltpu.async_copy(
                scores_hbm.at[idxs], out_v.at[5, dst], sem))
            for col in range(4):
                copies.append(pltpu.async_copy(
                    boxes_hbm.at[col].at[idxs], out_v.at[1 + col, dst], sem))
            for col in range(width_j):
                copies.append(pltpu.async_copy(
                    joints_hbm.at[col].at[idxs], out_v.at[6 + col, dst], sem))
        for cp in copies:
            cp.wait()
        pltpu.sync_copy(out_v, out_hbm.at[:, pl.ds(base, b_per_w)])

    return gather_kernel


def kernel(pred_boxes, pred_scores, pred_joints, selected_indexes):
    b, n = pred_boxes.shape[0], pred_boxes.shape[1]
    s = selected_indexes.shape[0]
    width_j = pred_joints.shape[2] * pred_joints.shape[3]

    info = plsc.get_sparse_core_info()
    nw = info.num_cores * info.num_subcores
    chunk = 80
    n_chunk = 2
    b_per_w = n_chunk * chunk
    s_pad = nw * b_per_w

    boxes_t = pred_boxes.transpose(2, 0, 1).reshape(4, b * n)
    scores_f = pred_scores.reshape(b * n)
    joints_t = pred_joints.transpose(2, 3, 0, 1).reshape(width_j, b * n)
    bcol = jnp.zeros((s_pad,), jnp.int32).at[:s].set(selected_indexes[:, 0])
    xcol = jnp.zeros((s_pad,), jnp.int32).at[:s].set(selected_indexes[:, 2])

    fn = _build_gather(n, width_j, s_pad, b_per_w, n_chunk, chunk,
                       info.num_cores)
    out_t = fn(bcol, xcol, boxes_t, scores_f, joints_t)
    return out_t[:, :s].T
